# Initial kernel scaffold; baseline (speedup 1.0000x reference)
import jax
import jax.numpy as jnp
from jax.experimental import pallas as pl


def kernel(vidx, pos, deg, W_vidx, W_pos, W_deg):
    raise NotImplementedError("write your pallas kernel here")


# speedup vs baseline: 5.3334x; 5.3334x over previous
import functools
import jax
import jax.numpy as jnp
from jax import lax
from jax.experimental import pallas as pl
from jax.experimental.pallas import tpu as pltpu, tpu_sc as plsc

V_DIM = 32
NC, NS = 2, 16
NW = NC * NS


@functools.lru_cache(maxsize=None)
def _make_sc_kernel(N, C, nchunk):
    per_w = N // NW
    mesh = plsc.VectorSubcoreMesh(core_axis_name="c", subcore_axis_name="s")

    @functools.partial(
        pl.kernel,
        out_type=jax.ShapeDtypeStruct((N, V_DIM), jnp.float32),
        mesh=mesh,
        scratch_types=[
            pltpu.VMEM((nchunk, C), jnp.int32),
            pltpu.VMEM((nchunk, C), jnp.int32),
            pltpu.VMEM((nchunk, C), jnp.int32),
            pltpu.VMEM((C, V_DIM), jnp.float32),
            pltpu.SemaphoreType.DMA,
        ],
        compiler_params=pltpu.CompilerParams(use_tc_tiling_on_sc=False),
    )
    def k(vidx_hbm, pos_hbm, deg_hbm, Wv, Wp, Wd, out_hbm, iv, ip, idg, rows, sem):
        wid = lax.axis_index("s") * NC + lax.axis_index("c")
        base = wid * per_w
        pltpu.sync_copy(vidx_hbm.at[wid], iv)
        pltpu.sync_copy(pos_hbm.at[wid], ip)
        pltpu.sync_copy(deg_hbm.at[wid], idg)

        def chunk(j, carry):
            pltpu.async_copy(Wv.at[iv.at[j]], rows, sem).wait()
            cp = pltpu.async_copy(Wp.at[ip.at[j]], rows, sem, add=True)
            cd = pltpu.async_copy(Wd.at[idg.at[j]], rows, sem, add=True)
            cp.wait()
            cd.wait()
            pltpu.sync_copy(rows, out_hbm.at[pl.ds(base + j * C, C)])
            return carry

        lax.fori_loop(0, nchunk, chunk, 0)

    return k


def kernel(vidx, pos, deg, W_vidx, W_pos, W_deg):
    B, L = vidx.shape
    N = B * L
    C = 128
    nchunk = N // (NW * C)
    iv = vidx.reshape(NW, nchunk, C).astype(jnp.int32)
    ip = pos.reshape(NW, nchunk, C).astype(jnp.int32)
    idg = deg.reshape(NW, nchunk, C).astype(jnp.int32)
    out = _make_sc_kernel(N, C, nchunk)(iv, ip, idg, W_vidx, W_pos, W_deg)
    return out.reshape(B, L, V_DIM)


# speedup vs baseline: 6.3465x; 1.1899x over previous
import functools
import jax
import jax.numpy as jnp
from jax import lax
from jax.experimental import pallas as pl
from jax.experimental.pallas import tpu as pltpu, tpu_sc as plsc

V_DIM = 32
NC, NS = 2, 16
NW = NC * NS


@functools.lru_cache(maxsize=None)
def _make_sc_kernel(N, C, K, nchunk):
    per_w = N // NW
    S = C * K
    nsuper = per_w // S
    mesh = plsc.VectorSubcoreMesh(core_axis_name="c", subcore_axis_name="s")

    @functools.partial(
        pl.kernel,
        out_type=jax.ShapeDtypeStruct((N, V_DIM), jnp.float32),
        mesh=mesh,
        scratch_types=[
            pltpu.VMEM((nchunk, C), jnp.int32),
            pltpu.VMEM((nchunk, C), jnp.int32),
            pltpu.VMEM((nchunk, C), jnp.int32),
            pltpu.VMEM((2, S, V_DIM), jnp.float32),
            pltpu.SemaphoreType.DMA,
            pltpu.SemaphoreType.DMA,
            pltpu.SemaphoreType.DMA,
            pltpu.SemaphoreType.DMA,
        ],
        compiler_params=pltpu.CompilerParams(use_tc_tiling_on_sc=False),
    )
    def k(vidx_hbm, pos_hbm, deg_hbm, Wv, Wp, Wd, out_hbm,
          iv, ip, idg, rows, sg0, sg1, sw0, sw1):
        wid = lax.axis_index("s") * NC + lax.axis_index("c")
        base = wid * per_w
        pltpu.sync_copy(vidx_hbm.at[wid], iv)
        pltpu.sync_copy(pos_hbm.at[wid], ip)
        pltpu.sync_copy(deg_hbm.at[wid], idg)
        sg = (sg0, sg1)
        sw = (sw0, sw1)

        def do_super(s, p):
            @pl.when(s >= 2)
            def _():
                pltpu.make_async_copy(
                    rows.at[p], out_hbm.at[pl.ds(base, S)], sw[p]).wait()
            buf = rows.at[p]
            gs = []
            for t in range(K):
                c = s * K + t
                gs.append(pltpu.async_copy(
                    Wv.at[iv.at[c]], buf.at[pl.ds(t * C, C)], sg[p]))
            for g in gs:
                g.wait()
            adds = []
            for t in range(K):
                c = s * K + t
                dst = buf.at[pl.ds(t * C, C)]
                adds.append(pltpu.async_copy(Wp.at[ip.at[c]], dst, sg[p], add=True))
                adds.append(pltpu.async_copy(Wd.at[idg.at[c]], dst, sg[p], add=True))
            for a in adds:
                a.wait()
            pltpu.async_copy(buf, out_hbm.at[pl.ds(base + s * S, S)], sw[p])

        def round_(g, carry):
            do_super(2 * g, 0)
            do_super(2 * g + 1, 1)
            return carry

        lax.fori_loop(0, nsuper // 2, round_, 0)
        for p in range(2):
            pltpu.make_async_copy(
                rows.at[p], out_hbm.at[pl.ds(base, S)], sw[p]).wait()

    return k


def kernel(vidx, pos, deg, W_vidx, W_pos, W_deg):
    B, L = vidx.shape
    N = B * L
    C = 128
    nchunk = N // (NW * C)
    iv = vidx.reshape(NW, nchunk, C).astype(jnp.int32)
    ip = pos.reshape(NW, nchunk, C).astype(jnp.int32)
    idg = deg.reshape(NW, nchunk, C).astype(jnp.int32)
    out = _make_sc_kernel(N, C, 4, nchunk)(iv, ip, idg, W_vidx, W_pos, W_deg)
    return out.reshape(B, L, V_DIM)
